# baseline (speedup 1.0000x reference)
import functools

import jax
import jax.numpy as jnp
from jax.experimental import pallas as pl
from jax.experimental.pallas import tpu as pltpu

BJORCK_ITER = 5
F8 = jnp.float8_e4m3fn
F4 = jnp.float4_e2m1fn


def _bjorck(w):
    w = w / (jnp.sqrt(jnp.sum(w * w)) + 1e-12)
    for _ in range(BJORCK_ITER):
        g = jax.lax.dot_general(w, w, (((0,), (0,)), ((), ())),
                                preferred_element_type=jnp.float32)
        w = 1.5 * w - 0.5 * jax.lax.dot_general(
            w, g, (((1,), (0,)), ((), ())),
            preferred_element_type=jnp.float32)
    return w


def _k1_body(na, x_ref, w0_ref, b_ref, adj_ref,
             h_ref, stats_ref, m4_ref, scale_ref,
             w0s_ref, t0s_ref):
    i = pl.program_id(0)
    rbt = x_ref.shape[0]

    @pl.when(i == 0)
    def _():
        w0s_ref[...] = _bjorck(w0_ref[...]).astype(jnp.bfloat16)

    @pl.when(i < na)
    def _():
        t = jnp.dot(x_ref[...].astype(jnp.bfloat16), w0s_ref[...],
                    preferred_element_type=jnp.float32)
        t0s_ref[pl.ds(i * rbt, rbt), :] = t.astype(jnp.bfloat16)

    @pl.when(i >= na)
    def _():
        a = adj_ref[...]
        mf = jnp.where(a != 0.0, 1.0, 0.0)
        m4_ref[...] = mf.astype(F4)
        c = jnp.sum(mf, axis=1, keepdims=True)
        scale = 1.0 / jnp.maximum(c, 1.0)
        scale_ref[...] = scale
        p = jnp.dot(mf.astype(jnp.bfloat16), t0s_ref[...],
                    preferred_element_type=jnp.float32)
        h = p * scale + b_ref[...]
        h_ref[...] = h.astype(F8)
        s0 = jnp.sum(h, axis=0, keepdims=True)
        s1 = jnp.sum(h * h, axis=0, keepdims=True)
        f = h.shape[1]
        st = jnp.concatenate([s0, s1, jnp.zeros((6, f), jnp.float32)], axis=0)

        @pl.when(i == na)
        def _():
            stats_ref[...] = st

        @pl.when(i > na)
        def _():
            stats_ref[...] = stats_ref[...] + st


def _k2_body(n, na, final, h_ref, stats_ref, g_ref, bb_ref, w_ref,
             m4_ref, scale_ref, b_ref, out_ref, statso_ref,
             ws_ref, ts_ref):
    i = pl.program_id(0)
    rbt = h_ref.shape[0]

    @pl.when(i == 0)
    def _():
        ws_ref[...] = _bjorck(w_ref[...]).astype(jnp.bfloat16)

    @pl.when(i < na)
    def _():
        st = stats_ref[...]
        mean = st[0:1, :] / n
        var = st[1:2, :] / n - mean * mean
        hh = h_ref[...].astype(jnp.float32)
        xn = (hh - mean) * jax.lax.rsqrt(var + 1e-5) * g_ref[...] + bb_ref[...]
        xn = jnp.maximum(xn, 0.0)
        t = jnp.dot(xn.astype(jnp.bfloat16), ws_ref[...],
                    preferred_element_type=jnp.float32)
        ts_ref[pl.ds(i * rbt, rbt), :] = t.astype(F8)

    @pl.when(i >= na)
    def _():
        p = jnp.dot(m4_ref[...], ts_ref[...], preferred_element_type=jnp.float32)
        v = p * scale_ref[...] + b_ref[...]
        if final:
            m = jnp.max(v, axis=1, keepdims=True)
            e = jnp.exp(v - m)
            s = jnp.sum(e, axis=1, keepdims=True)
            out_ref[...] = v - m - jnp.log(s)
        else:
            out_ref[...] = v.astype(F8)
            s0 = jnp.sum(v, axis=0, keepdims=True)
            s1 = jnp.sum(v * v, axis=0, keepdims=True)
            f = v.shape[1]
            st = jnp.concatenate(
                [s0, s1, jnp.zeros((6, f), jnp.float32)], axis=0)

            @pl.when(i == na)
            def _():
                statso_ref[...] = st

            @pl.when(i > na)
            def _():
                statso_ref[...] = statso_ref[...] + st


def kernel(x, adj, W0, b0, g0, bb0, W1, b1, g1, bb1, W2, b2):
    n, f_in = x.shape
    h_dim = W0.shape[1]
    c_dim = W2.shape[1]
    rb0 = 400 if n % 400 == 0 else n
    rba = 1000 if n % 1000 == 0 else n
    rbt = 2000 if n % 2000 == 0 else n
    na = n // rbt
    n0 = n // rb0
    naa = n // rba
    f32 = jnp.float32
    bf16 = jnp.bfloat16
    seq = pltpu.CompilerParams(dimension_semantics=("arbitrary",))

    h0, st0, mask4, scale = pl.pallas_call(
        functools.partial(_k1_body, na),
        grid=(na + n0,),
        in_specs=[
            pl.BlockSpec((rbt, f_in), lambda i: (jnp.minimum(i, na - 1), 0)),
            pl.BlockSpec((f_in, h_dim), lambda i: (0, 0)),
            pl.BlockSpec((1, h_dim), lambda i: (0, 0)),
            pl.BlockSpec((rb0, n), lambda i: (jnp.maximum(i - na, 0), 0)),
        ],
        out_specs=[
            pl.BlockSpec((rb0, h_dim), lambda i: (jnp.maximum(i - na, 0), 0)),
            pl.BlockSpec((8, h_dim), lambda i: (0, 0)),
            pl.BlockSpec((rb0, n), lambda i: (jnp.maximum(i - na, 0), 0)),
            pl.BlockSpec((rb0, 1), lambda i: (jnp.maximum(i - na, 0), 0)),
        ],
        out_shape=[
            jax.ShapeDtypeStruct((n, h_dim), F8),
            jax.ShapeDtypeStruct((8, h_dim), f32),
            jax.ShapeDtypeStruct((n, n), F4),
            jax.ShapeDtypeStruct((n, 1), f32),
        ],
        scratch_shapes=[
            pltpu.VMEM((f_in, h_dim), bf16),
            pltpu.VMEM((n, h_dim), bf16),
        ],
        compiler_params=seq,
    )(x, W0, b0.reshape(1, h_dim), adj)

    def layer(h, stats, g, bb, w, b, f_out, final):
        f_h = h.shape[1]
        return pl.pallas_call(
            functools.partial(_k2_body, float(n), na, final),
            grid=(na + naa,),
            in_specs=[
                pl.BlockSpec((rbt, f_h),
                             lambda i: (jnp.minimum(i, na - 1), 0)),
                pl.BlockSpec((8, f_h), lambda i: (0, 0)),
                pl.BlockSpec((1, f_h), lambda i: (0, 0)),
                pl.BlockSpec((1, f_h), lambda i: (0, 0)),
                pl.BlockSpec((f_h, f_out), lambda i: (0, 0)),
                pl.BlockSpec((rba, n), lambda i: (jnp.maximum(i - na, 0), 0)),
                pl.BlockSpec((rba, 1), lambda i: (jnp.maximum(i - na, 0), 0)),
                pl.BlockSpec((1, f_out), lambda i: (0, 0)),
            ],
            out_specs=[
                pl.BlockSpec((rba, f_out),
                             lambda i: (jnp.maximum(i - na, 0), 0)),
                pl.BlockSpec((8, f_out), lambda i: (0, 0)),
            ],
            out_shape=[
                jax.ShapeDtypeStruct((n, f_out), f32 if final else F8),
                jax.ShapeDtypeStruct((8, f_out), f32),
            ],
            scratch_shapes=[
                pltpu.VMEM((f_h, f_out), bf16),
                pltpu.VMEM((n, f_out), F8),
            ],
            compiler_params=seq,
        )(h, stats, g.reshape(1, -1), bb.reshape(1, -1), w,
          mask4, scale, b.reshape(1, f_out))

    h1, st1 = layer(h0, st0, g0, bb0, W1, b1, h_dim, False)
    out, _ = layer(h1, st1, g1, bb1, W2, b2, c_dim, True)
    return out
